# baseline (speedup 1.0000x reference)
import functools

import jax
import jax.numpy as jnp
from jax import lax
from jax.experimental import pallas as pl
from jax.experimental.pallas import tpu as pltpu
from jax.experimental.pallas import tpu_sc as plsc

N_ENT = 100000
N_REL = 1000
D = 64
W = 128
B = 16384
MULT = 20.0

NC = 2
NS = 16
L = 16
NW = NC * NS
BPW = B // NW
CH = 128
NCHUNK = BPW // CH
NG = CH // L

EBLK = 512


def _ident():
    r = lax.broadcasted_iota(jnp.int32, (D, D), 0)
    c = lax.broadcasted_iota(jnp.int32, (D, D), 1)
    return jnp.where(r == c, 1.0, 0.0).astype(jnp.float32)


_DN_T = (((0,), (0,)), ((), ()))


def _mxu_t(x, ident):
    return lax.dot_general(x, ident, _DN_T, preferred_element_type=jnp.float32)


def _prep_e_body(et_ref, ett_ref, out_ref):
    ident = _ident()
    out_ref[:, 0:D] = _mxu_t(et_ref[...], ident)
    out_ref[:, D:W] = _mxu_t(ett_ref[...], ident)


def _prep_r_body(rt_ref, rhtt_ref, rttt_ref, rp_ref, rhtt_out_ref):
    ident = _ident()
    r = _mxu_t(rt_ref[...], ident)
    rp_ref[:, 0:D] = r
    rp_ref[:, D:W] = jnp.zeros_like(r)
    rhtt_out_ref[:, 0:D] = _mxu_t(rhtt_ref[...], ident)
    rhtt_out_ref[:, D:W] = _mxu_t(rttt_ref[...], ident)


_prep_e = pl.pallas_call(
    _prep_e_body,
    grid=(pl.cdiv(N_ENT, EBLK),),
    in_specs=[
        pl.BlockSpec((D, EBLK), lambda i: (0, i)),
        pl.BlockSpec((D, EBLK), lambda i: (0, i)),
    ],
    out_specs=pl.BlockSpec((EBLK, W), lambda i: (i, 0)),
    out_shape=jax.ShapeDtypeStruct((N_ENT, W), jnp.float32),
)

_prep_r = pl.pallas_call(
    _prep_r_body,
    out_shape=[
        jax.ShapeDtypeStruct((N_REL, W), jnp.float32),
        jax.ShapeDtypeStruct((N_REL, W), jnp.float32),
    ],
)

_mesh = plsc.VectorSubcoreMesh(core_axis_name="c", subcore_axis_name="s")


@functools.partial(
    pl.kernel,
    out_type=jax.ShapeDtypeStruct((B,), jnp.float32),
    mesh=_mesh,
    compiler_params=pltpu.CompilerParams(
        needs_layout_passes=False, use_tc_tiling_on_sc=True),
    scratch_types=[
        pltpu.VMEM((CH,), jnp.int32),
        pltpu.VMEM((CH,), jnp.int32),
        pltpu.VMEM((CH,), jnp.int32),
        pltpu.VMEM((CH, W), jnp.float32),
        pltpu.VMEM((CH, W), jnp.float32),
        pltpu.VMEM((CH, W), jnp.float32),
        pltpu.VMEM((CH, W), jnp.float32),
        pltpu.VMEM((CH,), jnp.float32),
        pltpu.SemaphoreType.DMA,
    ],
)
def _sc_score(s_hbm, r_hbm, o_hbm, ee_hbm, rp_hbm, rhtt_hbm,
              out_hbm,
              sidx, ridx, oidx, srow, orow, rrow, rtrow, outv, sem):
    wid = lax.axis_index("s") * NC + lax.axis_index("c")

    def chunk_body(c, carry):
        base = pl.multiple_of(wid * BPW + c * CH, CH)
        pltpu.sync_copy(s_hbm.at[pl.ds(base, CH)], sidx)
        pltpu.sync_copy(r_hbm.at[pl.ds(base, CH)], ridx)
        pltpu.sync_copy(o_hbm.at[pl.ds(base, CH)], oidx)
        cps = [
            pltpu.async_copy(ee_hbm.at[sidx], srow, sem),
            pltpu.async_copy(ee_hbm.at[oidx], orow, sem),
            pltpu.async_copy(rp_hbm.at[ridx], rrow, sem),
            pltpu.async_copy(rhtt_hbm.at[ridx], rtrow, sem),
        ]
        for cp in cps:
            cp.wait()

        lane = lax.iota(jnp.int32, 16)
        for g in range(NG):
            tvec = lane + g * L

            def dim_body(d, accs):
                b_acc, h_acc, t_acc = accs
                dv = (lane + d) & 63
                dv2 = dv + 64
                s_e = plsc.load_gather(srow, [tvec, dv])
                s_t = plsc.load_gather(srow, [tvec, dv2])
                o_e = plsc.load_gather(orow, [tvec, dv])
                o_t = plsc.load_gather(orow, [tvec, dv2])
                r_e = plsc.load_gather(rrow, [tvec, dv])
                r_h = plsc.load_gather(rtrow, [tvec, dv])
                r_t = plsc.load_gather(rtrow, [tvec, dv2])
                return (b_acc + s_e * r_e * o_e,
                        h_acc + s_t * r_h,
                        t_acc + o_t * r_t)

            z = jnp.zeros((L,), jnp.float32)
            b_acc, h_acc, t_acc = lax.fori_loop(0, D, dim_body, (z, z, z))
            res = (MULT
                   / (1.0 + jnp.exp(-b_acc))
                   / (1.0 + jnp.exp(-h_acc))
                   / (1.0 + jnp.exp(-t_acc)))
            outv[pl.ds(g * L, L)] = res

        pltpu.sync_copy(outv, out_hbm.at[pl.ds(base, CH)])
        return carry

    lax.fori_loop(0, NCHUNK, chunk_body, 0)


def kernel(s, r, o, E, R, E_t, R_ht, R_tt):
    ee = _prep_e(E.T, E_t.T)
    rp, rhtt = _prep_r(R.T, R_ht.T, R_tt.T)
    return _sc_score(s, r, o, ee, rp, rhtt)


# speedup vs baseline: 1.6763x; 1.6763x over previous
import functools

import jax
import jax.numpy as jnp
from jax import lax
from jax.experimental import pallas as pl
from jax.experimental.pallas import tpu as pltpu
from jax.experimental.pallas import tpu_sc as plsc

N_ENT = 100000
N_REL = 1000
D = 64
W = 128
B = 16384
MULT = 20.0

NC = 2
NS = 16
L = 16
NW = NC * NS
BPW = B // NW
CH = 128
NCHUNK = BPW // CH
NG = CH // L

EBLK = 2048


def _ident():
    r = lax.broadcasted_iota(jnp.int32, (D, D), 0)
    c = lax.broadcasted_iota(jnp.int32, (D, D), 1)
    return jnp.where(r == c, 1.0, 0.0).astype(jnp.float32)


_DN_T = (((0,), (0,)), ((), ()))


def _mxu_t(x, ident):
    return lax.dot_general(x, ident, _DN_T, preferred_element_type=jnp.float32)


def _prep_e_body(et_ref, ett_ref, out_ref):
    ident = _ident()
    out_ref[:, 0:D] = _mxu_t(et_ref[...], ident)
    out_ref[:, D:W] = _mxu_t(ett_ref[...], ident)


def _prep_r_body(rt_ref, rhtt_ref, rttt_ref, rp_ref, rhtt_out_ref):
    ident = _ident()
    r = _mxu_t(rt_ref[...], ident)
    rp_ref[:, 0:D] = r
    rp_ref[:, D:W] = jnp.zeros_like(r)
    rhtt_out_ref[:, 0:D] = _mxu_t(rhtt_ref[...], ident)
    rhtt_out_ref[:, D:W] = _mxu_t(rttt_ref[...], ident)


_prep_e = pl.pallas_call(
    _prep_e_body,
    compiler_params=pltpu.CompilerParams(fuse_transposed_lhs_in_matmul=True),
    grid=(pl.cdiv(N_ENT, EBLK),),
    in_specs=[
        pl.BlockSpec((D, EBLK), lambda i: (0, i)),
        pl.BlockSpec((D, EBLK), lambda i: (0, i)),
    ],
    out_specs=pl.BlockSpec((EBLK, W), lambda i: (i, 0)),
    out_shape=jax.ShapeDtypeStruct((N_ENT, W), jnp.float32),
)

_prep_r = pl.pallas_call(
    _prep_r_body,
    compiler_params=pltpu.CompilerParams(fuse_transposed_lhs_in_matmul=True),
    out_shape=[
        jax.ShapeDtypeStruct((N_REL, W), jnp.float32),
        jax.ShapeDtypeStruct((N_REL, W), jnp.float32),
    ],
)

_mesh = plsc.VectorSubcoreMesh(core_axis_name="c", subcore_axis_name="s")


@functools.partial(
    pl.kernel,
    out_type=jax.ShapeDtypeStruct((B,), jnp.float32),
    mesh=_mesh,
    compiler_params=pltpu.CompilerParams(
        needs_layout_passes=False, use_tc_tiling_on_sc=True),
    scratch_types=[
        pltpu.VMEM((CH,), jnp.int32),
        pltpu.VMEM((CH,), jnp.int32),
        pltpu.VMEM((CH,), jnp.int32),
        pltpu.VMEM((CH, W), jnp.float32),
        pltpu.VMEM((CH, W), jnp.float32),
        pltpu.VMEM((CH, W), jnp.float32),
        pltpu.VMEM((CH, W), jnp.float32),
        pltpu.VMEM((CH,), jnp.float32),
        pltpu.SemaphoreType.DMA,
    ],
)
def _sc_score(s_hbm, r_hbm, o_hbm, ee_hbm, rp_hbm, rhtt_hbm,
              out_hbm,
              sidx, ridx, oidx, srow, orow, rrow, rtrow, outv, sem):
    wid = lax.axis_index("s") * NC + lax.axis_index("c")

    def chunk_body(c, carry):
        base = pl.multiple_of(wid * BPW + c * CH, CH)
        pltpu.sync_copy(s_hbm.at[pl.ds(base, CH)], sidx)
        pltpu.sync_copy(r_hbm.at[pl.ds(base, CH)], ridx)
        pltpu.sync_copy(o_hbm.at[pl.ds(base, CH)], oidx)
        cps = [
            pltpu.async_copy(ee_hbm.at[sidx], srow, sem),
            pltpu.async_copy(ee_hbm.at[oidx], orow, sem),
            pltpu.async_copy(rp_hbm.at[ridx], rrow, sem),
            pltpu.async_copy(rhtt_hbm.at[ridx], rtrow, sem),
        ]
        for cp in cps:
            cp.wait()

        lane = lax.iota(jnp.int32, 16)
        for g in range(NG):
            tvec = lane + g * L

            def dim_body(d, accs):
                b_acc, h_acc, t_acc = accs
                dv = (lane + d) & 63
                dv2 = dv + 64
                s_e = plsc.load_gather(srow, [tvec, dv])
                s_t = plsc.load_gather(srow, [tvec, dv2])
                o_e = plsc.load_gather(orow, [tvec, dv])
                o_t = plsc.load_gather(orow, [tvec, dv2])
                r_e = plsc.load_gather(rrow, [tvec, dv])
                r_h = plsc.load_gather(rtrow, [tvec, dv])
                r_t = plsc.load_gather(rtrow, [tvec, dv2])
                return (b_acc + s_e * r_e * o_e,
                        h_acc + s_t * r_h,
                        t_acc + o_t * r_t)

            z = jnp.zeros((L,), jnp.float32)
            b_acc, h_acc, t_acc = lax.fori_loop(0, D, dim_body, (z, z, z))
            res = (MULT
                   / (1.0 + jnp.exp(-b_acc))
                   / (1.0 + jnp.exp(-h_acc))
                   / (1.0 + jnp.exp(-t_acc)))
            outv[pl.ds(g * L, L)] = res

        pltpu.sync_copy(outv, out_hbm.at[pl.ds(base, CH)])
        return carry

    lax.fori_loop(0, NCHUNK, chunk_body, 0)


def kernel(s, r, o, E, R, E_t, R_ht, R_tt):
    ee = _prep_e(E.T, E_t.T)
    rp, rhtt = _prep_r(R.T, R_ht.T, R_tt.T)
    return _sc_score(s, r, o, ee, rp, rhtt)


# speedup vs baseline: 2.1230x; 1.2664x over previous
import functools

import jax
import jax.numpy as jnp
from jax import lax
from jax.experimental import pallas as pl
from jax.experimental.pallas import tpu as pltpu
from jax.experimental.pallas import tpu_sc as plsc

N_ENT = 100000
N_REL = 1000
D = 64
W = 128
B = 16384
MULT = 20.0

NC = 2
NS = 16
L = 16
NW = NC * NS
BPW = B // NW
CH = 64
NCHUNK = BPW // CH
NG = CH // L

EBLK = 4096


def _ident():
    r = lax.broadcasted_iota(jnp.int32, (D, D), 0)
    c = lax.broadcasted_iota(jnp.int32, (D, D), 1)
    return jnp.where(r == c, 1.0, 0.0).astype(jnp.float32)


_DN_T = (((0,), (0,)), ((), ()))


def _mxu_t(x, ident):
    return lax.dot_general(x, ident, _DN_T, preferred_element_type=jnp.float32)


def _prep_e_body(et_ref, ett_ref, out_ref):
    ident = _ident()
    out_ref[:, 0:D] = _mxu_t(et_ref[...], ident)
    out_ref[:, D:W] = _mxu_t(ett_ref[...], ident)


def _prep_r_body(rt_ref, rhtt_ref, rttt_ref, rp_ref, rhtt_out_ref):
    ident = _ident()
    r = _mxu_t(rt_ref[...], ident)
    rp_ref[:, 0:D] = r
    rp_ref[:, D:W] = jnp.zeros_like(r)
    rhtt_out_ref[:, 0:D] = _mxu_t(rhtt_ref[...], ident)
    rhtt_out_ref[:, D:W] = _mxu_t(rttt_ref[...], ident)


_prep_e = pl.pallas_call(
    _prep_e_body,
    compiler_params=pltpu.CompilerParams(fuse_transposed_lhs_in_matmul=True),
    grid=(pl.cdiv(N_ENT, EBLK),),
    in_specs=[
        pl.BlockSpec((D, EBLK), lambda i: (0, i)),
        pl.BlockSpec((D, EBLK), lambda i: (0, i)),
    ],
    out_specs=pl.BlockSpec((EBLK, W), lambda i: (i, 0)),
    out_shape=jax.ShapeDtypeStruct((N_ENT, W), jnp.float32),
)

_prep_r = pl.pallas_call(
    _prep_r_body,
    compiler_params=pltpu.CompilerParams(fuse_transposed_lhs_in_matmul=True),
    out_shape=[
        jax.ShapeDtypeStruct((N_REL, W), jnp.float32),
        jax.ShapeDtypeStruct((N_REL, W), jnp.float32),
    ],
)

_mesh = plsc.VectorSubcoreMesh(core_axis_name="c", subcore_axis_name="s")


@functools.partial(
    pl.kernel,
    out_type=jax.ShapeDtypeStruct((B,), jnp.float32),
    mesh=_mesh,
    compiler_params=pltpu.CompilerParams(
        needs_layout_passes=False, use_tc_tiling_on_sc=True),
    scratch_types=[
        pltpu.VMEM((BPW,), jnp.int32),
        pltpu.VMEM((BPW,), jnp.int32),
        pltpu.VMEM((BPW,), jnp.int32),
        pltpu.VMEM((CH, W), jnp.float32),
        pltpu.VMEM((CH, W), jnp.float32),
        pltpu.VMEM((CH, W), jnp.float32),
        pltpu.VMEM((CH, W), jnp.float32),
        pltpu.VMEM((CH, W), jnp.float32),
        pltpu.VMEM((CH, W), jnp.float32),
        pltpu.VMEM((CH, W), jnp.float32),
        pltpu.VMEM((CH, W), jnp.float32),
        pltpu.VMEM((BPW,), jnp.float32),
        pltpu.SemaphoreType.DMA,
        pltpu.SemaphoreType.DMA,
    ],
)
def _sc_score(s_hbm, r_hbm, o_hbm, ee_hbm, rp_hbm, rhtt_hbm,
              out_hbm,
              sidx, ridx, oidx,
              srow0, orow0, rrow0, rtrow0,
              srow1, orow1, rrow1, rtrow1,
              outv, sem0, sem1):
    wid = lax.axis_index("s") * NC + lax.axis_index("c")
    base = pl.multiple_of(wid * BPW, BPW)

    sets = ((srow0, orow0, rrow0, rtrow0, sem0),
            (srow1, orow1, rrow1, rtrow1, sem1))

    def fire(c, bset):
        srow, orow, rrow, rtrow, sem = bset
        off = pl.multiple_of(c * CH, CH)
        pltpu.async_copy(ee_hbm.at[sidx.at[pl.ds(off, CH)]], srow, sem)
        pltpu.async_copy(ee_hbm.at[oidx.at[pl.ds(off, CH)]], orow, sem)
        pltpu.async_copy(rp_hbm.at[ridx.at[pl.ds(off, CH)]], rrow, sem)
        pltpu.async_copy(rhtt_hbm.at[ridx.at[pl.ds(off, CH)]], rtrow, sem)

    def drain(c, bset):
        srow, orow, rrow, rtrow, sem = bset
        off = pl.multiple_of(c * CH, CH)
        pltpu.make_async_copy(ee_hbm.at[sidx.at[pl.ds(off, CH)]], srow, sem).wait()
        pltpu.make_async_copy(ee_hbm.at[oidx.at[pl.ds(off, CH)]], orow, sem).wait()
        pltpu.make_async_copy(rp_hbm.at[ridx.at[pl.ds(off, CH)]], rrow, sem).wait()
        pltpu.make_async_copy(rhtt_hbm.at[ridx.at[pl.ds(off, CH)]], rtrow, sem).wait()

    pltpu.sync_copy(s_hbm.at[pl.ds(base, BPW)], sidx)
    pltpu.sync_copy(r_hbm.at[pl.ds(base, BPW)], ridx)
    pltpu.sync_copy(o_hbm.at[pl.ds(base, BPW)], oidx)
    fire(0, sets[0])

    lane = lax.iota(jnp.int32, 16)

    def compute(c, bset):
        srow, orow, rrow, rtrow, _ = bset
        for g in range(NG):
            tvec = lane + g * L

            def dim_body(d, accs):
                b_acc, h_acc, t_acc = accs
                dv = (lane + d) & 63
                dv2 = dv + 64
                s_e = plsc.load_gather(srow, [tvec, dv])
                s_t = plsc.load_gather(srow, [tvec, dv2])
                o_e = plsc.load_gather(orow, [tvec, dv])
                o_t = plsc.load_gather(orow, [tvec, dv2])
                r_e = plsc.load_gather(rrow, [tvec, dv])
                r_h = plsc.load_gather(rtrow, [tvec, dv])
                r_t = plsc.load_gather(rtrow, [tvec, dv2])
                return (b_acc + s_e * r_e * o_e,
                        h_acc + s_t * r_h,
                        t_acc + o_t * r_t)

            z = jnp.zeros((L,), jnp.float32)
            b_acc, h_acc, t_acc = lax.fori_loop(0, D, dim_body, (z, z, z))
            res = (MULT
                   / (1.0 + jnp.exp(-b_acc))
                   / (1.0 + jnp.exp(-h_acc))
                   / (1.0 + jnp.exp(-t_acc)))
            outv[pl.ds(c * CH + g * L, L)] = res

    def pair_body(p, carry):
        for b in (0, 1):
            c = 2 * p + b

            @pl.when(c + 1 < NCHUNK)
            def _():
                fire(c + 1, sets[1 - b])

            drain(c, sets[b])
            compute(c, sets[b])
        return carry

    lax.fori_loop(0, NCHUNK // 2, pair_body, 0)
    pltpu.sync_copy(outv, out_hbm.at[pl.ds(base, BPW)])


def kernel(s, r, o, E, R, E_t, R_ht, R_tt):
    ee = _prep_e(E.T, E_t.T)
    rp, rhtt = _prep_r(R.T, R_ht.T, R_tt.T)
    return _sc_score(s, r, o, ee, rp, rhtt)


# speedup vs baseline: 2.2647x; 1.0668x over previous
import functools

import jax
import jax.numpy as jnp
from jax import lax
from jax.experimental import pallas as pl
from jax.experimental.pallas import tpu as pltpu
from jax.experimental.pallas import tpu_sc as plsc

N_ENT = 100000
N_REL = 1000
D = 64
W = 128
B = 16384
MULT = 20.0

NC = 2
NS = 16
L = 16
NW = NC * NS
BPW = B // NW
CH = 64
NCHUNK = BPW // CH
NG = CH // L

EBLK = 8192


def _ident():
    r = lax.broadcasted_iota(jnp.int32, (D, D), 0)
    c = lax.broadcasted_iota(jnp.int32, (D, D), 1)
    return jnp.where(r == c, 1.0, 0.0).astype(jnp.float32)


_DN_T = (((0,), (0,)), ((), ()))


def _mxu_t(x, ident):
    return lax.dot_general(x, ident, _DN_T, preferred_element_type=jnp.float32)


def _prep_body(et_ref, ett_ref, rt_ref, rhtt_ref, rttt_ref,
               out_ref, rp_ref, rhtt_out_ref):
    ident = _ident()
    out_ref[:, 0:D] = _mxu_t(et_ref[...], ident)
    out_ref[:, D:W] = _mxu_t(ett_ref[...], ident)

    @pl.when(pl.program_id(0) == 0)
    def _():
        r = _mxu_t(rt_ref[...], ident)
        rp_ref[:, 0:D] = r
        rp_ref[:, D:W] = jnp.zeros_like(r)
        rhtt_out_ref[:, 0:D] = _mxu_t(rhtt_ref[...], ident)
        rhtt_out_ref[:, D:W] = _mxu_t(rttt_ref[...], ident)


_prep = pl.pallas_call(
    _prep_body,
    compiler_params=pltpu.CompilerParams(fuse_transposed_lhs_in_matmul=True),
    grid=(pl.cdiv(N_ENT, EBLK),),
    in_specs=[
        pl.BlockSpec((D, EBLK), lambda i: (0, i)),
        pl.BlockSpec((D, EBLK), lambda i: (0, i)),
        pl.BlockSpec((D, N_REL), lambda i: (0, 0)),
        pl.BlockSpec((D, N_REL), lambda i: (0, 0)),
        pl.BlockSpec((D, N_REL), lambda i: (0, 0)),
    ],
    out_specs=[
        pl.BlockSpec((EBLK, W), lambda i: (i, 0)),
        pl.BlockSpec((N_REL, W), lambda i: (0, 0)),
        pl.BlockSpec((N_REL, W), lambda i: (0, 0)),
    ],
    out_shape=[
        jax.ShapeDtypeStruct((N_ENT, W), jnp.float32),
        jax.ShapeDtypeStruct((N_REL, W), jnp.float32),
        jax.ShapeDtypeStruct((N_REL, W), jnp.float32),
    ],
)

_mesh = plsc.VectorSubcoreMesh(core_axis_name="c", subcore_axis_name="s")


@functools.partial(
    pl.kernel,
    out_type=jax.ShapeDtypeStruct((B,), jnp.float32),
    mesh=_mesh,
    compiler_params=pltpu.CompilerParams(
        needs_layout_passes=False, use_tc_tiling_on_sc=True),
    scratch_types=[
        pltpu.VMEM((BPW,), jnp.int32),
        pltpu.VMEM((BPW,), jnp.int32),
        pltpu.VMEM((BPW,), jnp.int32),
        pltpu.VMEM((CH, W), jnp.float32),
        pltpu.VMEM((CH, W), jnp.float32),
        pltpu.VMEM((CH, W), jnp.float32),
        pltpu.VMEM((CH, W), jnp.float32),
        pltpu.VMEM((CH, W), jnp.float32),
        pltpu.VMEM((CH, W), jnp.float32),
        pltpu.VMEM((CH, W), jnp.float32),
        pltpu.VMEM((CH, W), jnp.float32),
        pltpu.VMEM((BPW,), jnp.float32),
        pltpu.SemaphoreType.DMA,
        pltpu.SemaphoreType.DMA,
    ],
)
def _sc_score(s_hbm, r_hbm, o_hbm, ee_hbm, rp_hbm, rhtt_hbm,
              out_hbm,
              sidx, ridx, oidx,
              srow0, orow0, rrow0, rtrow0,
              srow1, orow1, rrow1, rtrow1,
              outv, sem0, sem1):
    wid = lax.axis_index("s") * NC + lax.axis_index("c")
    base = pl.multiple_of(wid * BPW, BPW)

    sets = ((srow0, orow0, rrow0, rtrow0, sem0),
            (srow1, orow1, rrow1, rtrow1, sem1))

    def fire(c, bset):
        srow, orow, rrow, rtrow, sem = bset
        off = pl.multiple_of(c * CH, CH)
        pltpu.async_copy(ee_hbm.at[sidx.at[pl.ds(off, CH)]], srow, sem)
        pltpu.async_copy(ee_hbm.at[oidx.at[pl.ds(off, CH)]], orow, sem)
        pltpu.async_copy(rp_hbm.at[ridx.at[pl.ds(off, CH)]], rrow, sem)
        pltpu.async_copy(rhtt_hbm.at[ridx.at[pl.ds(off, CH)]], rtrow, sem)

    def drain(c, bset):
        srow, orow, rrow, rtrow, sem = bset
        off = pl.multiple_of(c * CH, CH)
        pltpu.make_async_copy(ee_hbm.at[sidx.at[pl.ds(off, CH)]], srow, sem).wait()
        pltpu.make_async_copy(ee_hbm.at[oidx.at[pl.ds(off, CH)]], orow, sem).wait()
        pltpu.make_async_copy(rp_hbm.at[ridx.at[pl.ds(off, CH)]], rrow, sem).wait()
        pltpu.make_async_copy(rhtt_hbm.at[ridx.at[pl.ds(off, CH)]], rtrow, sem).wait()

    pltpu.sync_copy(s_hbm.at[pl.ds(base, BPW)], sidx)
    pltpu.sync_copy(r_hbm.at[pl.ds(base, BPW)], ridx)
    pltpu.sync_copy(o_hbm.at[pl.ds(base, BPW)], oidx)
    fire(0, sets[0])

    lane = lax.iota(jnp.int32, 16)

    def compute(c, bset):
        srow, orow, rrow, rtrow, _ = bset
        for g in range(NG):
            tvec = lane + g * L

            def dim_body(d, accs):
                b_acc, h_acc, t_acc = accs
                for u in range(2):
                    dv = (lane + 2 * d + u) & 63
                    dv2 = dv + 64
                    s_e = plsc.load_gather(srow, [tvec, dv])
                    s_t = plsc.load_gather(srow, [tvec, dv2])
                    o_e = plsc.load_gather(orow, [tvec, dv])
                    o_t = plsc.load_gather(orow, [tvec, dv2])
                    r_e = plsc.load_gather(rrow, [tvec, dv])
                    r_h = plsc.load_gather(rtrow, [tvec, dv])
                    r_t = plsc.load_gather(rtrow, [tvec, dv2])
                    b_acc = b_acc + s_e * r_e * o_e
                    h_acc = h_acc + s_t * r_h
                    t_acc = t_acc + o_t * r_t
                return (b_acc, h_acc, t_acc)

            z = jnp.zeros((L,), jnp.float32)
            b_acc, h_acc, t_acc = lax.fori_loop(0, D // 2, dim_body, (z, z, z))
            res = (MULT
                   / (1.0 + jnp.exp(-b_acc))
                   / (1.0 + jnp.exp(-h_acc))
                   / (1.0 + jnp.exp(-t_acc)))
            outv[pl.ds(c * CH + g * L, L)] = res

    def pair_body(p, carry):
        for b in (0, 1):
            c = 2 * p + b

            @pl.when(c + 1 < NCHUNK)
            def _():
                fire(c + 1, sets[1 - b])

            drain(c, sets[b])
            compute(c, sets[b])
        return carry

    lax.fori_loop(0, NCHUNK // 2, pair_body, 0)
    pltpu.sync_copy(outv, out_hbm.at[pl.ds(base, BPW)])


def kernel(s, r, o, E, R, E_t, R_ht, R_tt):
    ee, rp, rhtt = _prep(E.T, E_t.T, R.T, R_ht.T, R_tt.T)
    return _sc_score(s, r, o, ee, rp, rhtt)


# speedup vs baseline: 2.3230x; 1.0257x over previous
import functools

import jax
import jax.numpy as jnp
from jax import lax
from jax.experimental import pallas as pl
from jax.experimental.pallas import tpu as pltpu
from jax.experimental.pallas import tpu_sc as plsc

N_ENT = 100000
N_REL = 1000
D = 64
W = 128
B = 16384
MULT = 20.0

NC = 2
NS = 16
L = 16
NW = NC * NS
BPW = B // NW
CH = 64
NCHUNK = BPW // CH
NG = CH // L

EBLK = 12544


def _ident():
    r = lax.broadcasted_iota(jnp.int32, (D, D), 0)
    c = lax.broadcasted_iota(jnp.int32, (D, D), 1)
    return jnp.where(r == c, 1.0, 0.0).astype(jnp.float32)


_DN_T = (((0,), (0,)), ((), ()))


def _mxu_t(x, ident):
    return lax.dot_general(x, ident, _DN_T, preferred_element_type=jnp.float32)


def _prep_body(et_ref, ett_ref, rt_ref, rhtt_ref, rttt_ref,
               out_ref, rp_ref, rhtt_out_ref):
    ident = _ident()
    out_ref[:, 0:D] = _mxu_t(et_ref[...], ident)
    out_ref[:, D:W] = _mxu_t(ett_ref[...], ident)

    @pl.when(pl.program_id(0) == 0)
    def _():
        r = _mxu_t(rt_ref[...], ident)
        rp_ref[:, 0:D] = r
        rp_ref[:, D:W] = jnp.zeros_like(r)
        rhtt_out_ref[:, 0:D] = _mxu_t(rhtt_ref[...], ident)
        rhtt_out_ref[:, D:W] = _mxu_t(rttt_ref[...], ident)


_prep = pl.pallas_call(
    _prep_body,
    compiler_params=pltpu.CompilerParams(fuse_transposed_lhs_in_matmul=True),
    grid=(pl.cdiv(N_ENT, EBLK),),
    in_specs=[
        pl.BlockSpec((D, EBLK), lambda i: (0, i)),
        pl.BlockSpec((D, EBLK), lambda i: (0, i)),
        pl.BlockSpec((D, N_REL), lambda i: (0, 0)),
        pl.BlockSpec((D, N_REL), lambda i: (0, 0)),
        pl.BlockSpec((D, N_REL), lambda i: (0, 0)),
    ],
    out_specs=[
        pl.BlockSpec((EBLK, W), lambda i: (i, 0)),
        pl.BlockSpec((N_REL, W), lambda i: (0, 0)),
        pl.BlockSpec((N_REL, W), lambda i: (0, 0)),
    ],
    out_shape=[
        jax.ShapeDtypeStruct((N_ENT, W), jnp.float32),
        jax.ShapeDtypeStruct((N_REL, W), jnp.float32),
        jax.ShapeDtypeStruct((N_REL, W), jnp.float32),
    ],
)

_mesh = plsc.VectorSubcoreMesh(core_axis_name="c", subcore_axis_name="s")


@functools.partial(
    pl.kernel,
    out_type=jax.ShapeDtypeStruct((B,), jnp.float32),
    mesh=_mesh,
    compiler_params=pltpu.CompilerParams(
        needs_layout_passes=False, use_tc_tiling_on_sc=True),
    scratch_types=[
        pltpu.VMEM((BPW,), jnp.int32),
        pltpu.VMEM((BPW,), jnp.int32),
        pltpu.VMEM((BPW,), jnp.int32),
        pltpu.VMEM((CH, W), jnp.float32),
        pltpu.VMEM((CH, W), jnp.float32),
        pltpu.VMEM((CH, W), jnp.float32),
        pltpu.VMEM((CH, W), jnp.float32),
        pltpu.VMEM((CH, W), jnp.float32),
        pltpu.VMEM((CH, W), jnp.float32),
        pltpu.VMEM((CH, W), jnp.float32),
        pltpu.VMEM((CH, W), jnp.float32),
        pltpu.VMEM((BPW,), jnp.float32),
        pltpu.SemaphoreType.DMA,
        pltpu.SemaphoreType.DMA,
    ],
)
def _sc_score(s_hbm, r_hbm, o_hbm, ee_hbm, rp_hbm, rhtt_hbm,
              out_hbm,
              sidx, ridx, oidx,
              srow0, orow0, rrow0, rtrow0,
              srow1, orow1, rrow1, rtrow1,
              outv, sem0, sem1):
    wid = lax.axis_index("s") * NC + lax.axis_index("c")
    base = pl.multiple_of(wid * BPW, BPW)

    sets = ((srow0, orow0, rrow0, rtrow0, sem0),
            (srow1, orow1, rrow1, rtrow1, sem1))

    def fire(c, bset):
        srow, orow, rrow, rtrow, sem = bset
        off = pl.multiple_of(c * CH, CH)
        pltpu.async_copy(ee_hbm.at[sidx.at[pl.ds(off, CH)]], srow, sem)
        pltpu.async_copy(ee_hbm.at[oidx.at[pl.ds(off, CH)]], orow, sem)
        pltpu.async_copy(rp_hbm.at[ridx.at[pl.ds(off, CH)]], rrow, sem)
        pltpu.async_copy(rhtt_hbm.at[ridx.at[pl.ds(off, CH)]], rtrow, sem)

    def drain(c, bset):
        srow, orow, rrow, rtrow, sem = bset
        off = pl.multiple_of(c * CH, CH)
        pltpu.make_async_copy(ee_hbm.at[sidx.at[pl.ds(off, CH)]], srow, sem).wait()
        pltpu.make_async_copy(ee_hbm.at[oidx.at[pl.ds(off, CH)]], orow, sem).wait()
        pltpu.make_async_copy(rp_hbm.at[ridx.at[pl.ds(off, CH)]], rrow, sem).wait()
        pltpu.make_async_copy(rhtt_hbm.at[ridx.at[pl.ds(off, CH)]], rtrow, sem).wait()

    pltpu.sync_copy(s_hbm.at[pl.ds(base, BPW)], sidx)
    pltpu.sync_copy(r_hbm.at[pl.ds(base, BPW)], ridx)
    pltpu.sync_copy(o_hbm.at[pl.ds(base, BPW)], oidx)
    fire(0, sets[0])

    lane = lax.iota(jnp.int32, 16)

    def compute(c, bset):
        srow, orow, rrow, rtrow, _ = bset
        for g in range(NG):
            tvec = lane + g * L

            def dim_body(d, accs):
                accs = list(accs)
                for u in range(2):
                    dv = (lane + 2 * d + u) & 63
                    dv2 = dv + 64
                    s_e = plsc.load_gather(srow, [tvec, dv])
                    s_t = plsc.load_gather(srow, [tvec, dv2])
                    o_e = plsc.load_gather(orow, [tvec, dv])
                    o_t = plsc.load_gather(orow, [tvec, dv2])
                    r_e = plsc.load_gather(rrow, [tvec, dv])
                    r_h = plsc.load_gather(rtrow, [tvec, dv])
                    r_t = plsc.load_gather(rtrow, [tvec, dv2])
                    accs[3 * u + 0] = accs[3 * u + 0] + s_e * r_e * o_e
                    accs[3 * u + 1] = accs[3 * u + 1] + s_t * r_h
                    accs[3 * u + 2] = accs[3 * u + 2] + o_t * r_t
                return tuple(accs)

            z = jnp.zeros((L,), jnp.float32)
            acc6 = lax.fori_loop(0, D // 2, dim_body, (z,) * 6)
            b_acc = acc6[0] + acc6[3]
            h_acc = acc6[1] + acc6[4]
            t_acc = acc6[2] + acc6[5]
            res = (MULT
                   / (1.0 + jnp.exp(-b_acc))
                   / (1.0 + jnp.exp(-h_acc))
                   / (1.0 + jnp.exp(-t_acc)))
            outv[pl.ds(c * CH + g * L, L)] = res

    def pair_body(p, carry):
        for b in (0, 1):
            c = 2 * p + b

            @pl.when(c + 1 < NCHUNK)
            def _():
                fire(c + 1, sets[1 - b])

            drain(c, sets[b])
            compute(c, sets[b])
        return carry

    lax.fori_loop(0, NCHUNK // 2, pair_body, 0)
    pltpu.sync_copy(outv, out_hbm.at[pl.ds(base, BPW)])


def kernel(s, r, o, E, R, E_t, R_ht, R_tt):
    ee, rp, rhtt = _prep(E.T, E_t.T, R.T, R_ht.T, R_tt.T)
    return _sc_score(s, r, o, ee, rp, rhtt)
